# baseline (speedup 1.0000x reference)
import jax
import jax.numpy as jnp
from jax.experimental import pallas as pl
from jax.experimental.pallas import tpu as pltpu

_PACKED_ROWS = 250000
_R = 10000


def _probe_body(x_ref, m_ref, o_ref):
    y = jnp.dot(x_ref[...], m_ref[...], preferred_element_type=jnp.float32)
    o_ref[0:8, 0:4] = y[0:8, :]


def kernel(item_ids, table, W, b):
    m = jnp.kron(jnp.eye(4, dtype=jnp.float32), W.reshape(32, 1))
    s = pl.pallas_call(
        _probe_body,
        grid=(_PACKED_ROWS // _R,),
        in_specs=[
            pl.BlockSpec((_R, 128), lambda i: (i, 0)),
            pl.BlockSpec((128, 4), lambda i: (0, 0)),
        ],
        out_specs=pl.BlockSpec((8, 128), lambda i: (i, 0)),
        out_shape=jax.ShapeDtypeStruct((_PACKED_ROWS // _R * 8, 128), jnp.float32),
    )(table.reshape(_PACKED_ROWS, 128), m)
    return jnp.broadcast_to(jnp.sum(s), (16384, 50, 1)).astype(jnp.float32)


# speedup vs baseline: 4.0321x; 4.0321x over previous
import functools

import jax
import jax.numpy as jnp
from jax import lax
from jax.experimental import pallas as pl
from jax.experimental.pallas import tpu as pltpu
from jax.experimental.pallas import tpu_sc as plsc

_NUM_ITEMS = 1000000
_EMBED = 32
_BLK = 65536


def _proj_body(xt_ref, w_ref, b_ref, o_ref):
    o_ref[...] = jnp.sum(xt_ref[...] * w_ref[...], axis=0) + b_ref[0]


def _project(table_t, w_col, b):
    return pl.pallas_call(
        _proj_body,
        grid=(pl.cdiv(_NUM_ITEMS, _BLK),),
        in_specs=[
            pl.BlockSpec((_EMBED, _BLK), lambda i: (0, i)),
            pl.BlockSpec((_EMBED, 1), lambda i: (0, 0)),
            pl.BlockSpec(memory_space=pltpu.SMEM),
        ],
        out_specs=pl.BlockSpec((_BLK,), lambda i: (i,)),
        out_shape=jax.ShapeDtypeStruct((_NUM_ITEMS,), jnp.float32),
    )(table_t, w_col, b)


@functools.cache
def _make_gather(num_elems):
    info = plsc.get_sparse_core_info()
    nc, ns = info.num_cores, info.num_subcores
    nw = nc * ns
    per_w = num_elems // nw
    assert per_w * nw == num_elems and per_w % 8 == 0
    mesh = plsc.VectorSubcoreMesh(core_axis_name="c", subcore_axis_name="s")

    @functools.partial(
        pl.kernel,
        mesh=mesh,
        out_type=jax.ShapeDtypeStruct((num_elems,), jnp.float32),
        scratch_types=[
            pltpu.VMEM((per_w,), jnp.int32),
            pltpu.VMEM((per_w,), jnp.float32),
            pltpu.SemaphoreType.DMA,
        ],
    )
    def gather_k(proj_hbm, idx_hbm, out_hbm, idx_v, vals_v, sem):
        wid = lax.axis_index("s") * nc + lax.axis_index("c")
        base = wid * per_w
        pltpu.sync_copy(idx_hbm.at[pl.ds(base, per_w)], idx_v)
        pltpu.async_copy(proj_hbm.at[idx_v], vals_v, sem).wait()
        pltpu.sync_copy(vals_v, out_hbm.at[pl.ds(base, per_w)])

    return gather_k


def kernel(item_ids, table, W, b):
    bsz, hist = item_ids.shape
    num_elems = bsz * hist
    proj = _project(table.T, W.reshape(_EMBED, 1), b)
    flat = _make_gather(num_elems)(
        proj, item_ids.reshape(num_elems).astype(jnp.int32)
    )
    return flat.reshape(bsz, hist, 1)


# speedup vs baseline: 4.0324x; 1.0001x over previous
import functools

import jax
import jax.numpy as jnp
from jax import lax
from jax.experimental import pallas as pl
from jax.experimental.pallas import tpu as pltpu
from jax.experimental.pallas import tpu_sc as plsc

_NUM_ITEMS = 1000000
_EMBED = 32
_BLK = 32768
_NS = 2
_LAST = (_NUM_ITEMS - 1) // _BLK


def _proj_body(x0_ref, x1_ref, w_ref, b_ref, o_ref):
    w = w_ref[...]
    bias = b_ref[0]
    o_ref[0 * _BLK : 1 * _BLK] = jnp.sum(x0_ref[...] * w, axis=0) + bias
    o_ref[1 * _BLK : 2 * _BLK] = jnp.sum(x1_ref[...] * w, axis=0) + bias


def _project(table_t, w_col, b):
    def _in_spec(k):
        return pl.BlockSpec(
            (_EMBED, _BLK), lambda i: (0, jnp.minimum(i * _NS + k, _LAST))
        )

    return pl.pallas_call(
        _proj_body,
        grid=(pl.cdiv(_NUM_ITEMS, _BLK * _NS),),
        in_specs=[
            _in_spec(0),
            _in_spec(1),
            pl.BlockSpec((_EMBED, 1), lambda i: (0, 0)),
            pl.BlockSpec(memory_space=pltpu.SMEM),
        ],
        out_specs=pl.BlockSpec((_BLK * _NS,), lambda i: (i,)),
        out_shape=jax.ShapeDtypeStruct((_NUM_ITEMS,), jnp.float32),
    )(table_t, table_t, w_col, b)


@functools.cache
def _make_gather(num_elems):
    info = plsc.get_sparse_core_info()
    nc, ns = info.num_cores, info.num_subcores
    nw = nc * ns
    per_w = num_elems // nw
    assert per_w * nw == num_elems and per_w % 8 == 0
    mesh = plsc.VectorSubcoreMesh(core_axis_name="c", subcore_axis_name="s")

    @functools.partial(
        pl.kernel,
        mesh=mesh,
        out_type=jax.ShapeDtypeStruct((num_elems,), jnp.float32),
        scratch_types=[
            pltpu.VMEM((per_w,), jnp.int32),
            pltpu.VMEM((per_w,), jnp.float32),
            pltpu.SemaphoreType.DMA,
        ],
    )
    def gather_k(proj_hbm, idx_hbm, out_hbm, idx_v, vals_v, sem):
        wid = lax.axis_index("s") * nc + lax.axis_index("c")
        base = wid * per_w
        pltpu.sync_copy(idx_hbm.at[pl.ds(base, per_w)], idx_v)
        pltpu.async_copy(proj_hbm.at[idx_v], vals_v, sem).wait()
        pltpu.sync_copy(vals_v, out_hbm.at[pl.ds(base, per_w)])

    return gather_k


def kernel(item_ids, table, W, b):
    bsz, hist = item_ids.shape
    num_elems = bsz * hist
    proj = _project(table.T, W.reshape(_EMBED, 1), b)
    flat = _make_gather(num_elems)(
        proj, item_ids.reshape(num_elems).astype(jnp.int32)
    )
    return flat.reshape(bsz, hist, 1)


# speedup vs baseline: 4.0840x; 1.0128x over previous
import functools

import jax
import jax.numpy as jnp
from jax import lax
from jax.experimental import pallas as pl
from jax.experimental.pallas import tpu as pltpu
from jax.experimental.pallas import tpu_sc as plsc

_NUM_ITEMS = 1000000
_EMBED = 32
_BLK = 131072


def _proj_body(xt_ref, w_ref, b_ref, o_ref):
    o_ref[...] = jnp.sum(xt_ref[...] * w_ref[...], axis=0) + b_ref[0]


def _project(table_t, w_col, b):
    return pl.pallas_call(
        _proj_body,
        grid=(pl.cdiv(_NUM_ITEMS, _BLK),),
        in_specs=[
            pl.BlockSpec((_EMBED, _BLK), lambda i: (0, i)),
            pl.BlockSpec((_EMBED, 1), lambda i: (0, 0)),
            pl.BlockSpec(memory_space=pltpu.SMEM),
        ],
        out_specs=pl.BlockSpec((_BLK,), lambda i: (i,)),
        out_shape=jax.ShapeDtypeStruct((_NUM_ITEMS,), jnp.float32),
    )(table_t, w_col, b)


@functools.cache
def _make_gather(num_elems):
    info = plsc.get_sparse_core_info()
    nc, ns = info.num_cores, info.num_subcores
    nw = nc * ns
    per_w = num_elems // nw
    assert per_w * nw == num_elems and per_w % 8 == 0
    mesh = plsc.VectorSubcoreMesh(core_axis_name="c", subcore_axis_name="s")

    @functools.partial(
        pl.kernel,
        mesh=mesh,
        out_type=jax.ShapeDtypeStruct((num_elems,), jnp.float32),
        scratch_types=[
            pltpu.VMEM((per_w,), jnp.int32),
            pltpu.VMEM((per_w,), jnp.float32),
            pltpu.SemaphoreType.DMA,
        ],
    )
    def gather_k(proj_hbm, idx_hbm, out_hbm, idx_v, vals_v, sem):
        wid = lax.axis_index("s") * nc + lax.axis_index("c")
        base = wid * per_w
        pltpu.sync_copy(idx_hbm.at[pl.ds(base, per_w)], idx_v)
        pltpu.async_copy(proj_hbm.at[idx_v], vals_v, sem).wait()
        pltpu.sync_copy(vals_v, out_hbm.at[pl.ds(base, per_w)])

    return gather_k


def kernel(item_ids, table, W, b):
    bsz, hist = item_ids.shape
    num_elems = bsz * hist
    proj = _project(table.T, W.reshape(_EMBED, 1), b)
    flat = _make_gather(num_elems)(
        proj, item_ids.reshape(num_elems).astype(jnp.int32)
    )
    return flat.reshape(bsz, hist, 1)
